# baseline (speedup 1.0000x reference)
import functools

import jax
import jax.numpy as jnp
from jax import lax
from jax.experimental import pallas as pl
from jax.experimental.pallas import tpu as pltpu
from jax.experimental.pallas import tpu_sc as plsc

N = 10000
D = 128

_NC = 2
_NS = 16
_NW = _NC * _NS
_CHUNK = 128
_CHUNKS_PER_W = 3
_BPW = _CHUNK * _CHUNKS_PER_W
_BPAD = _BPW * _NW


_STAGE = 640


def _sc_gather(table, x_pad):
    mesh = plsc.VectorSubcoreMesh(
        core_axis_name="c", subcore_axis_name="s",
        num_cores=_NC, num_subcores=_NS)

    @functools.partial(
        pl.kernel,
        out_type=jax.ShapeDtypeStruct((_BPAD, D), jnp.float32),
        mesh=mesh,
        scratch_types=[
            pltpu.VMEM((_CHUNKS_PER_W, _CHUNK), jnp.int32),
            pltpu.VMEM((_BPW, D), jnp.float32),
            pltpu.VMEM_SHARED((N, D), jnp.float32),
            pltpu.SemaphoreType.DMA,
        ],
    )
    def gather_kernel(table_hbm, idx_hbm, out_hbm, idx_v, rows_v, spmem, sem):
        cid = lax.axis_index("c")
        sid = lax.axis_index("s")
        wid = sid * _NC + cid
        @pl.when(sid < _NS - 1)
        def _stage_full():
            pltpu.sync_copy(
                table_hbm.at[pl.ds(sid * _STAGE, _STAGE)],
                spmem.at[pl.ds(sid * _STAGE, _STAGE)])

        @pl.when(sid == _NS - 1)
        def _stage_tail():
            pltpu.sync_copy(
                table_hbm.at[pl.ds((_NS - 1) * _STAGE, N - (_NS - 1) * _STAGE)],
                spmem.at[pl.ds((_NS - 1) * _STAGE, N - (_NS - 1) * _STAGE)])
        pltpu.sync_copy(idx_hbm.at[wid], idx_v)
        plsc.subcore_barrier()
        descs = []
        for j in range(_CHUNKS_PER_W):
            descs.append(pltpu.async_copy(
                spmem.at[idx_v.at[j]],
                rows_v.at[pl.ds(j * _CHUNK, _CHUNK)],
                sem))
        for d in descs:
            d.wait()
        pltpu.sync_copy(rows_v, out_hbm.at[pl.ds(wid * _BPW, _BPW)])

    return gather_kernel(table, x_pad.reshape(_NW, _CHUNKS_PER_W, _CHUNK))


_TR = 200


def _tc_body(g_ref, embf_ref, embt_ref, w_ref, b_ref, o_ref):
    m1 = lax.dot_general(
        g_ref[...], embf_ref[0:N, :],
        (((1,), (0,)), ((), ())),
        preferred_element_type=jnp.float32) + embt_ref[...]
    x1 = lax.dot_general(
        m1, w_ref[...],
        (((1,), (1,)), ((), ())),
        preferred_element_type=jnp.float32) + b_ref[...]
    x1 = jnp.where(x1 > 0, x1, 0.2 * x1)
    o_ref[:, 0:D] = embt_ref[...]
    o_ref[:, D:2 * D] = x1


def _tc_fused(G, emb_pad, W, b):
    grid = (N // _TR,)
    return pl.pallas_call(
        _tc_body,
        grid=grid,
        in_specs=[
            pl.BlockSpec((_TR, N), lambda i: (i, 0)),
            pl.BlockSpec((N, D), lambda i: (0, 0)),
            pl.BlockSpec((_TR, D), lambda i: (i, 0)),
            pl.BlockSpec((D, D), lambda i: (0, 0)),
            pl.BlockSpec((1, D), lambda i: (0, 0)),
        ],
        out_specs=pl.BlockSpec((_TR, 2 * D), lambda i: (i, 0)),
        out_shape=jax.ShapeDtypeStruct((N, 2 * D), jnp.float32),
    )(G, emb_pad, emb_pad, W, b.reshape(1, D))


def _sc_trivial(x_pad):
    mesh = plsc.VectorSubcoreMesh(
        core_axis_name="c", subcore_axis_name="s",
        num_cores=_NC, num_subcores=_NS)

    @functools.partial(
        pl.kernel,
        out_type=jax.ShapeDtypeStruct((_NW, _CHUNK), jnp.int32),
        mesh=mesh,
        scratch_types=[pltpu.VMEM((_CHUNK,), jnp.int32)],
    )
    def triv(idx_hbm, out_hbm, idx_v):
        wid = lax.axis_index("s") * _NC + lax.axis_index("c")
        pltpu.sync_copy(idx_hbm.at[wid], idx_v)
        pltpu.sync_copy(idx_v, out_hbm.at[wid])

    return triv(x_pad)


def kernel(x, G, table, W, b):
    x_pad = jnp.concatenate(
        [x.astype(jnp.int32), jnp.zeros((_BPAD - N,), jnp.int32)])
    emb_pad = _sc_gather(table, x_pad)
    return _tc_fused(G, emb_pad, W, b)
